# baseline (speedup 1.0000x reference)
import functools

import jax
import jax.numpy as jnp
from jax import lax
from jax.experimental import pallas as pl
from jax.experimental.pallas import tpu as pltpu
from jax.experimental.pallas import tpu_sc as plsc

NC = 2
NS = 16
NW = NC * NS
LANES = 16

L = 20
D = 32
CHUNK_BAGS = 32
CHUNK_IDX = CHUNK_BAGS * L
GATHER_W = 128
GATHERS = CHUNK_IDX // GATHER_W


def _emb_body(bags_per_w, chunks, idx_per_w,
              table_hbm, idx_hbm, out_hbm,
              idxw_v, rows0_v, rows1_v, stage0_v, stage1_v,
              sg0, sg1, so0, so1):
    wid = lax.axis_index("s") * NC + lax.axis_index("c")

    pltpu.sync_copy(idx_hbm.at[pl.ds(wid * idx_per_w, idx_per_w)],
                    idxw_v.at[pl.ds(0, idx_per_w)])

    def fire_gathers(c, rows_v, sem):
        for k in range(GATHERS):
            pltpu.async_copy(
                table_hbm.at[idxw_v.at[pl.ds(c * CHUNK_IDX + k * GATHER_W,
                                             GATHER_W)]],
                rows_v.at[pl.ds(k * GATHER_W, GATHER_W)],
                sem,
            )

    def drain_gathers(rows_v, sem):
        pltpu.make_async_copy(table_hbm.at[pl.ds(0, CHUNK_IDX)],
                              rows_v, sem).wait()

    def fire_out(c, stage_v, sem):
        out0 = wid * (bags_per_w * D) + c * (CHUNK_BAGS * D)
        pltpu.async_copy(stage_v, out_hbm.at[pl.ds(out0, CHUNK_BAGS * D)], sem)

    def drain_out(stage_v, sem):
        pltpu.make_async_copy(stage_v,
                              out_hbm.at[pl.ds(0, CHUNK_BAGS * D)],
                              sem).wait()

    def compute(c, rows_v, stage_v):
        base = c * CHUNK_IDX

        def bag_body(b, _):
            r = b * L
            m0 = jnp.where(idxw_v[pl.ds(base + r, LANES)] != 0,
                           jnp.float32(1.0), jnp.float32(0.0))
            m1 = jnp.where(idxw_v[pl.ds(base + r + LANES, LANES)] != 0,
                           jnp.float32(1.0), jnp.float32(0.0))
            acc0 = jnp.zeros((LANES,), jnp.float32)
            acc1 = jnp.zeros((LANES,), jnp.float32)
            for j in range(L):
                f = m0[j] if j < LANES else m1[j - LANES]
                acc0 = acc0 + rows_v[r + j, pl.ds(0, LANES)] * f
                acc1 = acc1 + rows_v[r + j, pl.ds(LANES, LANES)] * f
            stage_v[pl.ds(b * D, LANES)] = acc0
            stage_v[pl.ds(b * D + LANES, LANES)] = acc1
            return _

        lax.fori_loop(0, CHUNK_BAGS, bag_body, None)

    fire_gathers(0, rows0_v, sg0)

    def pair_body(cc, _):
        c0 = 2 * cc
        c1 = c0 + 1
        fire_gathers(c1, rows1_v, sg1)
        drain_gathers(rows0_v, sg0)
        pl.when(cc > 0)(lambda: drain_out(stage0_v, so0))
        compute(c0, rows0_v, stage0_v)
        fire_out(c0, stage0_v, so0)
        pl.when(c0 + 2 < chunks)(lambda: fire_gathers(c0 + 2, rows0_v, sg0))
        drain_gathers(rows1_v, sg1)
        pl.when(cc > 0)(lambda: drain_out(stage1_v, so1))
        compute(c1, rows1_v, stage1_v)
        fire_out(c1, stage1_v, so1)
        return _

    lax.fori_loop(0, chunks // 2, pair_body, None)
    drain_out(stage0_v, so0)
    drain_out(stage1_v, so1)


def _tc_table_relayout(table):
    V, d = table.shape
    W = 512
    G = (V + W - 1) // W
    v_pad = G * W

    def body(x_ref, o_ref):
        xt = x_ref[...].T
        y = xt.reshape(128, 4, 32)
        o_ref[...] = jnp.concatenate(
            [y[:, k, :] for k in range(4)], axis=1)

    out = pl.pallas_call(
        body,
        grid=(G,),
        in_specs=[pl.BlockSpec((d, W), lambda i: (0, i))],
        out_specs=pl.BlockSpec((W * d // 128, 128), lambda i: (i, 0)),
        out_shape=jax.ShapeDtypeStruct((v_pad * d // 128, 128), jnp.float32),
    )(table.T)
    return out.reshape(v_pad * d).reshape(v_pad, d)


def kernel(inputs, table):
    B, F, bag = inputs.shape
    V, d = table.shape
    assert bag == L and d == D
    bags = B * F
    assert bags % NW == 0
    bags_per_w = bags // NW
    assert bags_per_w % (2 * CHUNK_BAGS) == 0
    chunks = bags_per_w // CHUNK_BAGS
    idx_per_w = bags_per_w * L

    tbl_k = _tc_table_relayout(table)
    idx1d = inputs.reshape(bags * L).astype(jnp.int32)

    mesh = plsc.VectorSubcoreMesh(core_axis_name="c", subcore_axis_name="s")
    k = functools.partial(
        pl.kernel,
        mesh=mesh,
        compiler_params=pltpu.CompilerParams(use_tc_tiling_on_sc=False),
        out_type=jax.ShapeDtypeStruct((bags * D,), jnp.float32),
        scratch_types=[
            pltpu.VMEM((idx_per_w + LANES,), jnp.int32),
            pltpu.VMEM((CHUNK_IDX, D), jnp.float32),
            pltpu.VMEM((CHUNK_IDX, D), jnp.float32),
            pltpu.VMEM((CHUNK_BAGS * D,), jnp.float32),
            pltpu.VMEM((CHUNK_BAGS * D,), jnp.float32),
            pltpu.SemaphoreType.DMA,
            pltpu.SemaphoreType.DMA,
            pltpu.SemaphoreType.DMA,
            pltpu.SemaphoreType.DMA,
        ],
    )(functools.partial(_emb_body, bags_per_w, chunks, idx_per_w))

    out = k(tbl_k, idx1d)
    return out.reshape(B, F * D)


# speedup vs baseline: 2.3392x; 2.3392x over previous
import functools

import jax
import jax.numpy as jnp
from jax import lax
from jax.experimental import pallas as pl
from jax.experimental.pallas import tpu as pltpu
from jax.experimental.pallas import tpu_sc as plsc

NC = 2
NS = 16
NW = NC * NS
LANES = 16

L = 20
D = 32
CHUNK_BAGS = 32
CHUNK_IDX = CHUNK_BAGS * L
GATHER_W = 128
GATHERS = CHUNK_IDX // GATHER_W


def _emb_body(bags_per_w, chunks, idx_per_w,
              table_hbm, idx_hbm, out_hbm,
              idxw_v, rows0_v, rows1_v, stage0_v, stage1_v,
              sg0, sg1, so0, so1):
    wid = lax.axis_index("s") * NC + lax.axis_index("c")

    pltpu.sync_copy(idx_hbm.at[pl.ds(wid * idx_per_w, idx_per_w)],
                    idxw_v.at[pl.ds(0, idx_per_w)])

    def fire_gathers(c, rows_v, sem):
        for k in range(GATHERS):
            pltpu.async_copy(
                table_hbm.at[idxw_v.at[pl.ds(c * CHUNK_IDX + k * GATHER_W,
                                             GATHER_W)]],
                rows_v.at[pl.ds(k * GATHER_W, GATHER_W)],
                sem,
            )

    def drain_gathers(rows_v, sem):
        pltpu.make_async_copy(table_hbm.at[pl.ds(0, CHUNK_IDX)],
                              rows_v, sem).wait()

    def fire_out(c, stage_v, sem):
        out0 = wid * (bags_per_w * D) + c * (CHUNK_BAGS * D)
        pltpu.async_copy(stage_v, out_hbm.at[pl.ds(out0, CHUNK_BAGS * D)], sem)

    def drain_out(stage_v, sem):
        pltpu.make_async_copy(stage_v,
                              out_hbm.at[pl.ds(0, CHUNK_BAGS * D)],
                              sem).wait()

    def compute(c, rows_v, stage_v):
        base = c * CHUNK_IDX

        def bag_body(b, _):
            r = b * L
            m0 = jnp.where(idxw_v[pl.ds(base + r, LANES)] != 0,
                           jnp.float32(1.0), jnp.float32(0.0))
            m1 = jnp.where(idxw_v[pl.ds(base + r + LANES, LANES)] != 0,
                           jnp.float32(1.0), jnp.float32(0.0))
            acc0 = jnp.zeros((LANES,), jnp.float32)
            acc1 = jnp.zeros((LANES,), jnp.float32)
            for j in range(L):
                f = m0[j] if j < LANES else m1[j - LANES]
                acc0 = acc0 + rows_v[r + j, pl.ds(0, LANES)] * f
                acc1 = acc1 + rows_v[r + j, pl.ds(LANES, LANES)] * f
            stage_v[pl.ds(b * D, LANES)] = acc0
            stage_v[pl.ds(b * D + LANES, LANES)] = acc1
            return _

        lax.fori_loop(0, CHUNK_BAGS, bag_body, None)

    fire_gathers(0, rows0_v, sg0)

    def pair_body(cc, _):
        c0 = 2 * cc
        c1 = c0 + 1
        fire_gathers(c1, rows1_v, sg1)
        drain_gathers(rows0_v, sg0)
        pl.when(cc > 0)(lambda: drain_out(stage0_v, so0))
        compute(c0, rows0_v, stage0_v)
        fire_out(c0, stage0_v, so0)
        pl.when(c0 + 2 < chunks)(lambda: fire_gathers(c0 + 2, rows0_v, sg0))
        drain_gathers(rows1_v, sg1)
        pl.when(cc > 0)(lambda: drain_out(stage1_v, so1))
        compute(c1, rows1_v, stage1_v)
        fire_out(c1, stage1_v, so1)
        return _

    lax.fori_loop(0, chunks // 2, pair_body, None)
    drain_out(stage0_v, so0)
    drain_out(stage1_v, so1)


def _tc_table_relayout(table):
    V, d = table.shape
    W = 16384
    G = (V + W - 1) // W
    v_pad = G * W

    def body(x_ref, o_ref):
        xt = x_ref[...].T
        y = xt.reshape(W // 4, 4, 32)
        o_ref[...] = jnp.concatenate(
            [y[:, k, :] for k in range(4)], axis=1)

    out = pl.pallas_call(
        body,
        grid=(G,),
        in_specs=[pl.BlockSpec((d, W), lambda i: (0, i))],
        out_specs=pl.BlockSpec((W * d // 128, 128), lambda i: (i, 0)),
        out_shape=jax.ShapeDtypeStruct((v_pad * d // 128, 128), jnp.float32),
    )(table.T)
    return out.reshape(v_pad * d).reshape(v_pad, d)


def kernel(inputs, table):
    B, F, bag = inputs.shape
    V, d = table.shape
    assert bag == L and d == D
    bags = B * F
    assert bags % NW == 0
    bags_per_w = bags // NW
    assert bags_per_w % (2 * CHUNK_BAGS) == 0
    chunks = bags_per_w // CHUNK_BAGS
    idx_per_w = bags_per_w * L

    tbl_k = _tc_table_relayout(table)
    idx1d = inputs.reshape(bags * L).astype(jnp.int32)

    mesh = plsc.VectorSubcoreMesh(core_axis_name="c", subcore_axis_name="s")
    k = functools.partial(
        pl.kernel,
        mesh=mesh,
        compiler_params=pltpu.CompilerParams(use_tc_tiling_on_sc=False),
        out_type=jax.ShapeDtypeStruct((bags * D,), jnp.float32),
        scratch_types=[
            pltpu.VMEM((idx_per_w + LANES,), jnp.int32),
            pltpu.VMEM((CHUNK_IDX, D), jnp.float32),
            pltpu.VMEM((CHUNK_IDX, D), jnp.float32),
            pltpu.VMEM((CHUNK_BAGS * D,), jnp.float32),
            pltpu.VMEM((CHUNK_BAGS * D,), jnp.float32),
            pltpu.SemaphoreType.DMA,
            pltpu.SemaphoreType.DMA,
            pltpu.SemaphoreType.DMA,
            pltpu.SemaphoreType.DMA,
        ],
    )(functools.partial(_emb_body, bags_per_w, chunks, idx_per_w))

    out = k(tbl_k, idx1d)
    return out.reshape(B, F * D)


# speedup vs baseline: 2.3488x; 1.0041x over previous
import functools

import jax
import jax.numpy as jnp
from jax import lax
from jax.experimental import pallas as pl
from jax.experimental.pallas import tpu as pltpu
from jax.experimental.pallas import tpu_sc as plsc

NC = 2
NS = 16
NW = NC * NS
LANES = 16

L = 20
D = 32
CHUNK_BAGS = 32
CHUNK_IDX = CHUNK_BAGS * L
GATHER_W = 128
GATHERS = CHUNK_IDX // GATHER_W


def _emb_body(bags_per_w, chunks, idx_per_w,
              table_hbm, idx_hbm, out_hbm,
              idxw_v, rows0_v, rows1_v, stage0_v, stage1_v,
              sg0, sg1, so0, so1):
    wid = lax.axis_index("s") * NC + lax.axis_index("c")

    pltpu.sync_copy(idx_hbm.at[pl.ds(wid * idx_per_w, idx_per_w)],
                    idxw_v.at[pl.ds(0, idx_per_w)])

    def fire_gathers(c, rows_v, sem):
        for k in range(GATHERS):
            pltpu.async_copy(
                table_hbm.at[idxw_v.at[pl.ds(c * CHUNK_IDX + k * GATHER_W,
                                             GATHER_W)]],
                rows_v.at[pl.ds(k * GATHER_W, GATHER_W)],
                sem,
            )

    def drain_gathers(rows_v, sem):
        pltpu.make_async_copy(table_hbm.at[pl.ds(0, CHUNK_IDX)],
                              rows_v, sem).wait()

    def fire_out(c, stage_v, sem):
        out0 = wid * (bags_per_w * D) + c * (CHUNK_BAGS * D)
        pltpu.async_copy(stage_v, out_hbm.at[pl.ds(out0, CHUNK_BAGS * D)], sem)

    def drain_out(stage_v, sem):
        pltpu.make_async_copy(stage_v,
                              out_hbm.at[pl.ds(0, CHUNK_BAGS * D)],
                              sem).wait()

    def compute(c, rows_v, stage_v):
        base = c * CHUNK_IDX

        def bag_body(b, _):
            r = b * L
            m0 = jnp.where(idxw_v[pl.ds(base + r, LANES)] != 0,
                           jnp.float32(1.0), jnp.float32(0.0))
            m1 = jnp.where(idxw_v[pl.ds(base + r + LANES, LANES)] != 0,
                           jnp.float32(1.0), jnp.float32(0.0))
            acc0 = jnp.zeros((LANES,), jnp.float32)
            acc1 = jnp.zeros((LANES,), jnp.float32)
            for j in range(L):
                f = m0[j] if j < LANES else m1[j - LANES]
                acc0 = acc0 + rows_v[r + j, pl.ds(0, LANES)] * f
                acc1 = acc1 + rows_v[r + j, pl.ds(LANES, LANES)] * f
            stage_v[pl.ds(b * D, LANES)] = acc0
            stage_v[pl.ds(b * D + LANES, LANES)] = acc1
            return _

        lax.fori_loop(0, CHUNK_BAGS, bag_body, None)

    fire_gathers(0, rows0_v, sg0)

    def pair_body(cc, _):
        c0 = 2 * cc
        c1 = c0 + 1
        fire_gathers(c1, rows1_v, sg1)
        drain_gathers(rows0_v, sg0)
        pl.when(cc > 0)(lambda: drain_out(stage0_v, so0))
        compute(c0, rows0_v, stage0_v)
        fire_out(c0, stage0_v, so0)
        pl.when(c0 + 2 < chunks)(lambda: fire_gathers(c0 + 2, rows0_v, sg0))
        drain_gathers(rows1_v, sg1)
        pl.when(cc > 0)(lambda: drain_out(stage1_v, so1))
        compute(c1, rows1_v, stage1_v)
        fire_out(c1, stage1_v, so1)
        return _

    lax.fori_loop(0, chunks // 2, pair_body, None)
    drain_out(stage0_v, so0)
    drain_out(stage1_v, so1)


def _tc_table_relayout(table):
    V, d = table.shape
    W = 32768
    G = (V + W - 1) // W
    v_pad = G * W

    def body(x_ref, o_ref):
        xt = x_ref[...].T
        y = xt.reshape(W // 4, 4, 32)
        o_ref[...] = jnp.concatenate(
            [y[:, k, :] for k in range(4)], axis=1)

    out = pl.pallas_call(
        body,
        grid=(G,),
        in_specs=[pl.BlockSpec((d, W), lambda i: (0, i))],
        out_specs=pl.BlockSpec((W * d // 128, 128), lambda i: (i, 0)),
        out_shape=jax.ShapeDtypeStruct((v_pad * d // 128, 128), jnp.float32),
    )(table.T)
    return out.reshape(v_pad * d).reshape(v_pad, d)


def kernel(inputs, table):
    B, F, bag = inputs.shape
    V, d = table.shape
    assert bag == L and d == D
    bags = B * F
    assert bags % NW == 0
    bags_per_w = bags // NW
    assert bags_per_w % (2 * CHUNK_BAGS) == 0
    chunks = bags_per_w // CHUNK_BAGS
    idx_per_w = bags_per_w * L

    tbl_k = _tc_table_relayout(table)
    idx1d = inputs.reshape(bags * L).astype(jnp.int32)

    mesh = plsc.VectorSubcoreMesh(core_axis_name="c", subcore_axis_name="s")
    k = functools.partial(
        pl.kernel,
        mesh=mesh,
        compiler_params=pltpu.CompilerParams(use_tc_tiling_on_sc=False),
        out_type=jax.ShapeDtypeStruct((bags * D,), jnp.float32),
        scratch_types=[
            pltpu.VMEM((idx_per_w + LANES,), jnp.int32),
            pltpu.VMEM((CHUNK_IDX, D), jnp.float32),
            pltpu.VMEM((CHUNK_IDX, D), jnp.float32),
            pltpu.VMEM((CHUNK_BAGS * D,), jnp.float32),
            pltpu.VMEM((CHUNK_BAGS * D,), jnp.float32),
            pltpu.SemaphoreType.DMA,
            pltpu.SemaphoreType.DMA,
            pltpu.SemaphoreType.DMA,
            pltpu.SemaphoreType.DMA,
        ],
    )(functools.partial(_emb_body, bags_per_w, chunks, idx_per_w))

    out = k(tbl_k, idx1d)
    return out.reshape(B, F * D)


# speedup vs baseline: 2.4835x; 1.0574x over previous
import functools

import jax
import jax.numpy as jnp
from jax import lax
from jax.experimental import pallas as pl
from jax.experimental.pallas import tpu as pltpu
from jax.experimental.pallas import tpu_sc as plsc

NC = 2
NS = 16
NW = NC * NS
LANES = 16

L = 20
D = 32
CHUNK_BAGS = 32
CHUNK_IDX = CHUNK_BAGS * L
GATHER_W = 128
GATHERS = CHUNK_IDX // GATHER_W


def _emb_body(bags_per_w, chunks, idx_per_w,
              table_hbm, idx_hbm, out_hbm,
              idxw_v, rows0_v, rows1_v, stage0_v, stage1_v,
              sg0, sg1, so0, so1):
    wid = lax.axis_index("s") * NC + lax.axis_index("c")

    pltpu.sync_copy(idx_hbm.at[pl.ds(wid * idx_per_w, idx_per_w)],
                    idxw_v.at[pl.ds(0, idx_per_w)])

    def fire_gathers(c, rows_v, sem):
        for k in range(GATHERS):
            pltpu.async_copy(
                table_hbm.at[idxw_v.at[pl.ds(c * CHUNK_IDX + k * GATHER_W,
                                             GATHER_W)]],
                rows_v.at[pl.ds(k * GATHER_W, GATHER_W)],
                sem,
            )

    def drain_gathers(rows_v, sem):
        pltpu.make_async_copy(table_hbm.at[pl.ds(0, CHUNK_IDX)],
                              rows_v, sem).wait()

    def fire_out(c, stage_v, sem):
        out0 = wid * (bags_per_w * D) + c * (CHUNK_BAGS * D)
        pltpu.async_copy(stage_v, out_hbm.at[pl.ds(out0, CHUNK_BAGS * D)], sem)

    def drain_out(stage_v, sem):
        pltpu.make_async_copy(stage_v,
                              out_hbm.at[pl.ds(0, CHUNK_BAGS * D)],
                              sem).wait()

    def compute(c, rows_v, stage_v):
        base = c * CHUNK_IDX

        def bag_body(b, _):
            r = b * L
            m0 = jnp.where(idxw_v[pl.ds(base + r, LANES)] != 0,
                           jnp.float32(1.0), jnp.float32(0.0))
            m1 = jnp.where(idxw_v[pl.ds(base + r + LANES, LANES)] != 0,
                           jnp.float32(1.0), jnp.float32(0.0))
            acc0 = jnp.zeros((LANES,), jnp.float32)
            acc1 = jnp.zeros((LANES,), jnp.float32)
            for j in range(L):
                f = m0[j] if j < LANES else m1[j - LANES]
                acc0 = acc0 + rows_v[r + j, pl.ds(0, LANES)] * f
                acc1 = acc1 + rows_v[r + j, pl.ds(LANES, LANES)] * f
            stage_v[pl.ds(b * D, LANES)] = acc0
            stage_v[pl.ds(b * D + LANES, LANES)] = acc1
            return _

        lax.fori_loop(0, CHUNK_BAGS, bag_body, None)

    fire_gathers(0, rows0_v, sg0)

    def pair_body(cc, _):
        c0 = 2 * cc
        c1 = c0 + 1
        fire_gathers(c1, rows1_v, sg1)
        drain_gathers(rows0_v, sg0)
        pl.when(cc > 0)(lambda: drain_out(stage0_v, so0))
        compute(c0, rows0_v, stage0_v)
        fire_out(c0, stage0_v, so0)
        pl.when(c0 + 2 < chunks)(lambda: fire_gathers(c0 + 2, rows0_v, sg0))
        drain_gathers(rows1_v, sg1)
        pl.when(cc > 0)(lambda: drain_out(stage1_v, so1))
        compute(c1, rows1_v, stage1_v)
        fire_out(c1, stage1_v, so1)
        return _

    lax.fori_loop(0, chunks // 2, pair_body, None)
    drain_out(stage0_v, so0)
    drain_out(stage1_v, so1)


def _tc_table_relayout(table):
    V, d = table.shape
    W = 16384
    G = (V + W - 1) // W
    v_pad = G * W

    def body(x_ref, o_ref):
        xt = x_ref[...].T
        y = xt.reshape(W // 4, 4, 32)
        for k in range(4):
            o_ref[:, k * 32:(k + 1) * 32] = y[:, k, :]

    out = pl.pallas_call(
        body,
        grid=(G,),
        in_specs=[pl.BlockSpec((d, W), lambda i: (0, i))],
        out_specs=pl.BlockSpec((W * d // 128, 128), lambda i: (i, 0)),
        out_shape=jax.ShapeDtypeStruct((v_pad * d // 128, 128), jnp.float32),
    )(table.T)
    return out.reshape(v_pad * d).reshape(v_pad, d)


def kernel(inputs, table):
    B, F, bag = inputs.shape
    V, d = table.shape
    assert bag == L and d == D
    bags = B * F
    assert bags % NW == 0
    bags_per_w = bags // NW
    assert bags_per_w % (2 * CHUNK_BAGS) == 0
    chunks = bags_per_w // CHUNK_BAGS
    idx_per_w = bags_per_w * L

    tbl_k = _tc_table_relayout(table)
    idx1d = inputs.reshape(bags * L).astype(jnp.int32)

    mesh = plsc.VectorSubcoreMesh(core_axis_name="c", subcore_axis_name="s")
    k = functools.partial(
        pl.kernel,
        mesh=mesh,
        compiler_params=pltpu.CompilerParams(use_tc_tiling_on_sc=False),
        out_type=jax.ShapeDtypeStruct((bags * D,), jnp.float32),
        scratch_types=[
            pltpu.VMEM((idx_per_w + LANES,), jnp.int32),
            pltpu.VMEM((CHUNK_IDX, D), jnp.float32),
            pltpu.VMEM((CHUNK_IDX, D), jnp.float32),
            pltpu.VMEM((CHUNK_BAGS * D,), jnp.float32),
            pltpu.VMEM((CHUNK_BAGS * D,), jnp.float32),
            pltpu.SemaphoreType.DMA,
            pltpu.SemaphoreType.DMA,
            pltpu.SemaphoreType.DMA,
            pltpu.SemaphoreType.DMA,
        ],
    )(functools.partial(_emb_body, bags_per_w, chunks, idx_per_w))

    out = k(tbl_k, idx1d)
    return out.reshape(B, F * D)
